# baseline (speedup 1.0000x reference)
import jax
import jax.numpy as jnp
from jax.experimental import pallas as pl
from jax.experimental.pallas import tpu as pltpu

B = 16384
D = 8
H = 256
BB = 4096


def _body(idx_ref, gt_ref, ct_ref, w1_ref, b1_ref, w2_ref, b2_ref, out_ref,
          a_ref):
    @pl.when(pl.program_id(0) == 0)
    def _fold():
        gpad = jnp.concatenate([gt_ref[:], jnp.zeros((6, D), jnp.float32)], axis=0)
        cpad = jnp.concatenate([ct_ref[:], jnp.zeros((5, D), jnp.float32)], axis=0)
        w1 = w1_ref[:]
        parts = []
        for p in range(5):
            parts.append(jax.lax.dot(gpad, w1[8 * p:8 * p + 8, :],
                                     preferred_element_type=jnp.float32))
        for p in range(5):
            parts.append(jax.lax.dot(cpad, w1[40 + 8 * p:48 + 8 * p, :],
                                     preferred_element_type=jnp.float32))
        parts.append(jax.lax.dot(gpad, w1[80:88, :],
                                 preferred_element_type=jnp.float32))
        parts.append(jax.lax.dot(gpad, w1[88:96, :],
                                 preferred_element_type=jnp.float32))
        a_ref[:] = jnp.concatenate(parts, axis=0).astype(jnp.bfloat16)

    si = jax.lax.broadcasted_iota(jnp.int32, (32, BB), 0)
    tiles = []
    for s in range(10):
        row = idx_ref[s:s + 1, :]
        tiles.append(jnp.where(si == row, 1.0, 0.0))
    acc = jnp.zeros((32, BB), jnp.float32)
    for j in range(10):
        acc = acc + jnp.where(si == idx_ref[10 + j:11 + j, :], 0.1, 0.0)
    tiles.append(acc)
    acc = jnp.zeros((32, BB), jnp.float32)
    for j in range(10):
        acc = acc + jnp.where(si == idx_ref[20 + j:21 + j, :], 0.1, 0.0)
    tiles.append(acc)
    o_t = jnp.concatenate(tiles, axis=0).astype(jnp.bfloat16)

    h = jax.lax.dot_general(o_t, a_ref[:], (((0,), (0,)), ((), ())),
                            preferred_element_type=jnp.float32)
    h = jnp.maximum(h + b1_ref[:], 0.0).astype(jnp.bfloat16)
    out = jax.lax.dot(h, w2_ref[:], preferred_element_type=jnp.float32)
    out_ref[:] = out + b2_ref[:]


@jax.jit
def kernel(guess_indices, constraint_indices, presence_list, absent_list,
           guess_table, constraint_table, W1, b1, W2, b2):
    idx_t = jnp.concatenate([guess_indices, constraint_indices,
                             presence_list, absent_list], axis=1).astype(jnp.int32).T
    b1r = b1.reshape(1, H).astype(jnp.bfloat16)
    w2r = W2.astype(jnp.bfloat16)
    b2r = b2.reshape(1, 1)
    grid = (B // BB,)
    return pl.pallas_call(
        _body,
        grid=grid,
        in_specs=[
            pl.BlockSpec((30, BB), lambda i: (0, i)),
            pl.BlockSpec((26, D), lambda i: (0, 0)),
            pl.BlockSpec((27, D), lambda i: (0, 0)),
            pl.BlockSpec((96, H), lambda i: (0, 0)),
            pl.BlockSpec((1, H), lambda i: (0, 0)),
            pl.BlockSpec((H, 1), lambda i: (0, 0)),
            pl.BlockSpec((1, 1), lambda i: (0, 0)),
        ],
        out_specs=pl.BlockSpec((BB, 1), lambda i: (i, 0)),
        out_shape=jax.ShapeDtypeStruct((B, 1), jnp.float32),
        scratch_shapes=[pltpu.VMEM((384, H), jnp.bfloat16)],
    )(idx_t, guess_table, constraint_table, W1, b1r, w2r, b2r)
